# Initial kernel scaffold; baseline (speedup 1.0000x reference)
import jax
import jax.numpy as jnp
from jax.experimental import pallas as pl


def kernel(x):
    raise NotImplementedError("write your pallas kernel here")


# speedup vs baseline: 1.0297x; 1.0297x over previous
import functools

import jax
import jax.numpy as jnp
from jax import lax
from jax.experimental import pallas as pl
from jax.experimental.pallas import tpu as pltpu
from jax.experimental.pallas import tpu_sc as plsc

_N = 1000
_B = 16384
_NC = 2
_NS = 16
_NW = _NC * _NS
_RPW = _B // _NW
_CH = 64
_NCHUNK = _RPW // _CH
_CHW = _CH * _N
_L = 16

_mesh = plsc.VectorSubcoreMesh(core_axis_name="c", subcore_axis_name="s")


@functools.partial(
    pl.kernel,
    out_type=jax.ShapeDtypeStruct((_B * _N,), jnp.float32),
    mesh=_mesh,
    compiler_params=pltpu.CompilerParams(needs_layout_passes=False),
    scratch_types=[
        pltpu.VMEM((_RPW,), jnp.int32),
        pltpu.VMEM((_CHW,), jnp.float32),
        pltpu.VMEM((_CHW,), jnp.float32),
        pltpu.SemaphoreType.DMA,
        pltpu.SemaphoreType.DMA,
    ],
)
def _onehot_sc(x_hbm, out_hbm, idx_v, buf0, buf1, sem0, sem1):
    wid = lax.axis_index("s") * _NC + lax.axis_index("c")
    base_row = wid * _RPW

    pltpu.sync_copy(x_hbm.at[pl.ds(base_row, _RPW)], idx_v)

    zeros = jnp.zeros((_L,), jnp.float32)
    ones = jnp.ones((_L,), jnp.float32)
    iota = lax.iota(jnp.int32, _L)

    unroll = 8
    def zero_body(i, carry):
        for u in range(unroll):
            off = (i * unroll + u) * _L
            buf0[pl.ds(off, _L)] = zeros
            buf1[pl.ds(off, _L)] = zeros
        return carry
    lax.fori_loop(0, _CHW // _L // unroll, zero_body, 0)

    def scatter(buf, g, val):
        for k in range(_CH // _L):
            xv = idx_v[pl.ds(g * _CH + k * _L, _L)]
            flat = (iota + (k * _L)) * _N + xv
            plsc.store_scatter(buf, [flat], val)

    bufs = (buf0, buf1)
    sems = (sem0, sem1)
    copies = [None, None]
    for g in range(_NCHUNK):
        b = g % 2
        if copies[b] is not None:
            copies[b].wait()
            scatter(bufs[b], g - 2, zeros)
        scatter(bufs[b], g, ones)
        dst = out_hbm.at[pl.ds((base_row + g * _CH) * _N, _CHW)]
        copies[b] = pltpu.async_copy(bufs[b], dst, sems[b])
    copies[(_NCHUNK - 2) % 2].wait()
    copies[(_NCHUNK - 1) % 2].wait()


def kernel(x):
    out = _onehot_sc(x.astype(jnp.int32))
    return out.reshape(_B, _N)


# speedup vs baseline: 1.6750x; 1.6267x over previous
import functools

import jax
import jax.numpy as jnp
from jax import lax
from jax.experimental import pallas as pl
from jax.experimental.pallas import tpu as pltpu
from jax.experimental.pallas import tpu_sc as plsc

_N = 1000
_B = 16384
_NC = 2
_NS = 16
_NW = _NC * _NS
_RPW = _B // _NW
_CH = 32
_NCHUNK = _RPW // _CH
_L = 16

_mesh = plsc.VectorSubcoreMesh(core_axis_name="c", subcore_axis_name="s")

_ZCOLS = list(range(0, _N - _L + 1, _L))
if _ZCOLS[-1] != _N - _L:
    _ZCOLS.append(_N - _L)


@functools.partial(
    pl.kernel,
    out_type=jax.ShapeDtypeStruct((_B, _N), jnp.float32),
    mesh=_mesh,
    compiler_params=pltpu.CompilerParams(needs_layout_passes=False),
    scratch_types=[
        pltpu.VMEM((_RPW,), jnp.int32),
        pltpu.VMEM((_CH, _N), jnp.float32),
        pltpu.VMEM((_CH, _N), jnp.float32),
        pltpu.SemaphoreType.DMA,
        pltpu.SemaphoreType.DMA,
    ],
)
def _onehot_sc(x_hbm, out_hbm, idx_v, buf0, buf1, sem0, sem1):
    wid = lax.axis_index("s") * _NC + lax.axis_index("c")
    base_row = wid * _RPW

    pltpu.sync_copy(x_hbm.at[pl.ds(base_row, _RPW)], idx_v)

    zeros = jnp.zeros((_L,), jnp.float32)
    ones = jnp.ones((_L,), jnp.float32)
    iota = lax.iota(jnp.int32, _L)

    def zero_body(r, carry):
        for c in _ZCOLS:
            buf0[r, pl.ds(c, _L)] = zeros
            buf1[r, pl.ds(c, _L)] = zeros
        return carry
    lax.fori_loop(0, _CH, zero_body, 0)

    def scatter(buf, g, val):
        for k in range(_CH // _L):
            xv = idx_v[pl.ds(g * _CH + k * _L, _L)]
            rows = iota + (k * _L)
            plsc.store_scatter(buf, [rows, xv], val)

    bufs = (buf0, buf1)
    sems = (sem0, sem1)
    copies = [None, None]
    for g in range(_NCHUNK):
        b = g % 2
        if copies[b] is not None:
            copies[b].wait()
            scatter(bufs[b], g - 2, zeros)
        scatter(bufs[b], g, ones)
        dst = out_hbm.at[pl.ds(base_row + g * _CH, _CH)]
        copies[b] = pltpu.async_copy(bufs[b], dst, sems[b])
    copies[(_NCHUNK - 2) % 2].wait()
    copies[(_NCHUNK - 1) % 2].wait()


def kernel(x):
    return _onehot_sc(x.astype(jnp.int32))


# speedup vs baseline: 3.6875x; 2.2015x over previous
import functools

import jax
import jax.numpy as jnp
from jax import lax
from jax.experimental import pallas as pl
from jax.experimental.pallas import tpu as pltpu
from jax.experimental.pallas import tpu_sc as plsc

_N = 1000
_B = 16384
_NC = 2
_NS = 16
_NW = _NC * _NS
_BPW = _B // _NW
_CT = 128
_NCT = _BPW // _CT
_H0 = 504
_H1 = _N - _H0
_L = 16

_mesh = plsc.VectorSubcoreMesh(core_axis_name="c", subcore_axis_name="s")


@functools.partial(
    pl.kernel,
    out_type=jax.ShapeDtypeStruct((_N, _B), jnp.float32),
    mesh=_mesh,
    compiler_params=pltpu.CompilerParams(needs_layout_passes=False),
    scratch_types=[
        pltpu.VMEM((_BPW,), jnp.int32),
        pltpu.VMEM((_H0, _CT), jnp.float32),
        pltpu.VMEM((_H0, _CT), jnp.float32),
        pltpu.SemaphoreType.DMA,
        pltpu.SemaphoreType.DMA,
    ],
)
def _onehot_t_sc(x_hbm, out_hbm, idx_v, buf0, buf1, sem0, sem1):
    wid = lax.axis_index("s") * _NC + lax.axis_index("c")
    base_col = wid * _BPW

    pltpu.sync_copy(x_hbm.at[pl.ds(base_col, _BPW)], idx_v)

    zeros = jnp.zeros((_L,), jnp.float32)
    ones = jnp.ones((_L,), jnp.float32)
    iota = lax.iota(jnp.int32, _L)

    def zero_body(r, carry):
        for c in range(0, _CT, _L):
            buf0[r, pl.ds(c, _L)] = zeros
            buf1[r, pl.ds(c, _L)] = zeros
        return carry
    lax.fori_loop(0, _H0, zero_body, 0)

    def scatter(buf, ct, lo, hi, val):
        for k in range(_CT // _L):
            xv = idx_v[pl.ds(ct * _CT + k * _L, _L)]
            cols = iota + (k * _L)
            mask = (xv >= lo) & (xv < hi)
            plsc.store_scatter(buf, [xv - lo, cols], val, mask=mask)

    chunks = [(ct, h) for ct in range(_NCT) for h in range(2)]
    halves = ((0, _H0), (_H0, _N))
    bufs = (buf0, buf1)
    sems = (sem0, sem1)
    copies = [None, None]
    for i, (ct, h) in enumerate(chunks):
        b = i % 2
        lo, hi = halves[h]
        if copies[b] is not None:
            copies[b].wait()
            pct, ph = chunks[i - 2]
            plo, phi = halves[ph]
            scatter(bufs[b], pct, plo, phi, zeros)
        scatter(bufs[b], ct, lo, hi, ones)
        rows = hi - lo
        src = bufs[b] if rows == _H0 else bufs[b].at[pl.ds(0, rows)]
        dst = out_hbm.at[pl.ds(lo, rows), pl.ds(base_col + ct * _CT, _CT)]
        copies[b] = pltpu.async_copy(src, dst, sems[b])
    copies[0].wait()
    copies[1].wait()


def kernel(x):
    return _onehot_t_sc(x.astype(jnp.int32)).T


# speedup vs baseline: 3.8680x; 1.0490x over previous
import functools

import jax
import jax.numpy as jnp
from jax import lax
from jax.experimental import pallas as pl
from jax.experimental.pallas import tpu as pltpu
from jax.experimental.pallas import tpu_sc as plsc

_N = 1000
_B = 16384
_NC = 2
_NS = 16
_NW = _NC * _NS
_BPW = _B // _NW
_CT = 128
_NCT = _BPW // _CT
_H0 = 504
_H1 = _N - _H0
_L = 16

_mesh = plsc.VectorSubcoreMesh(core_axis_name="c", subcore_axis_name="s")


@functools.partial(
    pl.kernel,
    out_type=jax.ShapeDtypeStruct((_N, _B), jnp.float32),
    mesh=_mesh,
    compiler_params=pltpu.CompilerParams(needs_layout_passes=False),
    scratch_types=[
        pltpu.VMEM((_BPW,), jnp.int32),
        pltpu.VMEM((_H0, _CT), jnp.float32),
        pltpu.VMEM((_H0, _CT), jnp.float32),
        pltpu.SemaphoreType.DMA,
        pltpu.SemaphoreType.DMA,
    ],
)
def _onehot_t_sc(x_hbm, out_hbm, idx_v, buf0, buf1, sem0, sem1):
    wid = lax.axis_index("s") * _NC + lax.axis_index("c")
    base_col = wid * _BPW

    pltpu.sync_copy(x_hbm.at[pl.ds(base_col, _BPW)], idx_v)

    zeros = jnp.zeros((_L,), jnp.float32)
    ones = jnp.ones((_L,), jnp.float32)
    iota = lax.iota(jnp.int32, _L)

    def zero_fill(buf):
        def zero_body(r, carry):
            for c in range(0, _CT, _L):
                buf[r, pl.ds(c, _L)] = zeros
            return carry
        lax.fori_loop(0, _H0, zero_body, 0)

    def scatter(buf, ct, lo, hi, val):
        for k in range(_CT // _L):
            xv = idx_v[pl.ds(ct * _CT + k * _L, _L)]
            cols = iota + (k * _L)
            mask = (xv >= lo) & (xv < hi)
            plsc.store_scatter(buf, [xv - lo, cols], val, mask=mask)

    chunks = [(ct, h) for ct in range(_NCT) for h in range(2)]
    halves = ((0, _H0), (_H0, _N))
    bufs = (buf0, buf1)
    sems = (sem0, sem1)
    copies = [None, None]
    for i, (ct, h) in enumerate(chunks):
        b = i % 2
        lo, hi = halves[h]
        if i < 2:
            zero_fill(bufs[b])
        if copies[b] is not None:
            copies[b].wait()
            pct, ph = chunks[i - 2]
            plo, phi = halves[ph]
            scatter(bufs[b], pct, plo, phi, zeros)
        scatter(bufs[b], ct, lo, hi, ones)
        rows = hi - lo
        src = bufs[b] if rows == _H0 else bufs[b].at[pl.ds(0, rows)]
        dst = out_hbm.at[pl.ds(lo, rows), pl.ds(base_col + ct * _CT, _CT)]
        copies[b] = pltpu.async_copy(src, dst, sems[b])
    copies[0].wait()
    copies[1].wait()


def kernel(x):
    return _onehot_t_sc(x.astype(jnp.int32)).T
